# baseline (speedup 1.0000x reference)
import functools

import jax
import jax.numpy as jnp
from jax import lax
from jax.experimental import pallas as pl
from jax.experimental.pallas import tpu as pltpu
from jax.experimental.pallas import tpu_sc as plsc

N = 10000
E = 160000
D = 256
H = D // 2

NC = 2
NS = 16
L = 16
CHUNK = 128
NCHUNK = E // CHUNK
MAXK = -(-NCHUNK // NS)
NROW = 3
NIDX = 6
UNROLL = 6
TRIPS = -(-(MAXK + 2) // UNROLL)
ROWS_PER_TILE = 624
TAIL_ROWS = N - NS * ROWS_PER_TILE


@functools.cache
def _make_sc_gather_scatter():
    mesh = plsc.VectorSubcoreMesh(
        core_axis_name="c", subcore_axis_name="s",
        num_cores=NC, num_subcores=NS)

    @functools.partial(
        pl.kernel,
        out_type=jax.ShapeDtypeStruct((NC, N, H), jnp.float32),
        mesh=mesh,
        scratch_types=[
            pltpu.VMEM((NIDX, 2, CHUNK), jnp.int32),
            pltpu.VMEM((NROW, CHUNK, H), jnp.float32),
            pltpu.VMEM_SHARED((N, H), jnp.float32),
        ] + [pltpu.SemaphoreType.DMA] * (NIDX + 2 * NROW),
    )
    def sc_gather_scatter(xv, ei_hbm, out_hbm, idx_v, rows_v, h_sp, *sems):
        cid = lax.axis_index("c")
        tid = lax.axis_index("s")
        isem = sems[:NIDX]
        gsem = sems[NIDX:NIDX + NROW]
        ssem = sems[NIDX + NROW:]
        def unit(i):
            return i * 2 + cid

        def valid(j):
            return tid + NS * j < NCHUNK

        def off(j):
            return (tid + NS * j) * CHUNK

        def fire_idx(j, u):
            pltpu.async_copy(ei_hbm.at[0, pl.ds(off(j), CHUNK)],
                             idx_v.at[u % NIDX, 0], isem[u % NIDX])
            pltpu.async_copy(ei_hbm.at[1, pl.ds(off(j), CHUNK)],
                             idx_v.at[u % NIDX, 1], isem[u % NIDX])

        def drain_idx(j, u):
            pltpu.make_async_copy(ei_hbm.at[0, pl.ds(off(j), CHUNK)],
                                  idx_v.at[u % NIDX, 0], isem[u % NIDX]).wait()
            pltpu.make_async_copy(ei_hbm.at[1, pl.ds(off(j), CHUNK)],
                                  idx_v.at[u % NIDX, 1], isem[u % NIDX]).wait()
            for c in range(CHUNK // L):
                sl = pl.ds(c * L, L)
                idx_v[u % NIDX, 0, sl] = unit(idx_v[u % NIDX, 0, sl])

        def fire_gather(j, u):
            pltpu.async_copy(xv.at[idx_v.at[u % NIDX, 0]],
                             rows_v.at[u % NROW], gsem[u % NROW])

        def drain_gather(j, u):
            pltpu.make_async_copy(xv.at[idx_v.at[u % NIDX, 0]],
                                  rows_v.at[u % NROW], gsem[u % NROW]).wait()

        def fire_scatter(j, u):
            pltpu.async_copy(rows_v.at[u % NROW],
                             h_sp.at[idx_v.at[u % NIDX, 1]], ssem[u % NROW],
                             add=True)

        def drain_scatter(j, u):
            pltpu.make_async_copy(rows_v.at[u % NROW],
                                  h_sp.at[idx_v.at[u % NIDX, 1]],
                                  ssem[u % NROW]).wait()

        fire_idx(0, 0)
        fire_idx(1, 1)
        fire_idx(2, 2)

        r0 = tid * ROWS_PER_TILE
        n_blk = -(-ROWS_PER_TILE // CHUNK)
        for p in range(n_blk):
            base = r0 + p * CHUNK
            for c in range(CHUNK // L):
                i16 = jnp.minimum(base + c * L + lax.iota(jnp.int32, L), N - 1)
                idx_v[3, 0, pl.ds(c * L, L)] = unit(i16)
            pltpu.async_copy(xv.at[idx_v.at[3, 0]], rows_v.at[0],
                             gsem[0]).wait()
            nrows = min(CHUNK, ROWS_PER_TILE - p * CHUNK)
            pltpu.sync_copy(rows_v.at[0, pl.ds(0, nrows)],
                            h_sp.at[pl.ds(base, nrows)])

        @pl.when(tid == NS - 1)
        def _():
            t0 = NS * ROWS_PER_TILE
            for c in range(TAIL_ROWS // L):
                i16 = jnp.minimum(t0 + c * L + lax.iota(jnp.int32, L), N - 1)
                idx_v[3, 0, pl.ds(c * L, L)] = unit(i16)
            pltpu.async_copy(xv.at[idx_v.at[3, 0, pl.ds(0, TAIL_ROWS)]],
                             rows_v.at[0, pl.ds(0, TAIL_ROWS)],
                             gsem[0]).wait()
            pltpu.sync_copy(rows_v.at[0, pl.ds(0, TAIL_ROWS)],
                            h_sp.at[pl.ds(t0, TAIL_ROWS)])

        plsc.subcore_barrier()

        drain_idx(0, 0)
        fire_gather(0, 0)
        drain_idx(1, 1)
        fire_gather(1, 1)

        def outer(q, carry):
            for u in range(UNROLL):
                j = q * UNROLL + u
                pl.when((j >= 1) & valid(j - 1))(
                    functools.partial(drain_scatter, j - 1, u - 1))
                pl.when(valid(j + 2))(
                    functools.partial(drain_idx, j + 2, u + 2))
                pl.when(valid(j + 2))(
                    functools.partial(fire_gather, j + 2, u + 2))
                pl.when(valid(j + 3))(
                    functools.partial(fire_idx, j + 3, u + 3))
                pl.when(valid(j))(
                    functools.partial(drain_gather, j, u))
                pl.when(valid(j))(
                    functools.partial(fire_scatter, j, u))
            return carry

        lax.fori_loop(0, TRIPS, outer, 0)
        plsc.subcore_barrier()
        pltpu.sync_copy(h_sp.at[pl.ds(r0, ROWS_PER_TILE)],
                        out_hbm.at[cid, pl.ds(r0, ROWS_PER_TILE)])

        @pl.when(tid == NS - 1)
        def _():
            t0 = NS * ROWS_PER_TILE
            pltpu.sync_copy(h_sp.at[pl.ds(t0, TAIL_ROWS)],
                            out_hbm.at[cid, pl.ds(t0, TAIL_ROWS)])

    return sc_gather_scatter


ROWS_BLK = 1000


def _mlp_body(h2_ref, w1_ref, b1_ref, w2_ref, b2_ref, out_ref):
    h = jnp.concatenate([h2_ref[0], h2_ref[1]], axis=-1)
    z = lax.dot_general(h, w1_ref[...], (((1,), (1,)), ((), ())),
                        preferred_element_type=jnp.float32) + b1_ref[...]
    z = jnp.maximum(z, 0.0)
    out_ref[...] = lax.dot_general(z, w2_ref[...], (((1,), (1,)), ((), ())),
                                   preferred_element_type=jnp.float32) + b2_ref[...]


_mlp = pl.pallas_call(
    _mlp_body,
    grid=(N // ROWS_BLK,),
    in_specs=[
        pl.BlockSpec((NC, ROWS_BLK, H), lambda i: (0, i, 0)),
        pl.BlockSpec((D, D), lambda i: (0, 0)),
        pl.BlockSpec((1, D), lambda i: (0, 0)),
        pl.BlockSpec((D, D), lambda i: (0, 0)),
        pl.BlockSpec((1, D), lambda i: (0, 0)),
    ],
    out_specs=pl.BlockSpec((ROWS_BLK, D), lambda i: (i, 0)),
    out_shape=jax.ShapeDtypeStruct((N, D), jnp.float32),
)


def kernel(x, edge_index, W1, b1, W2, b2):
    x2 = x.reshape(NC * N, H)
    h2 = _make_sc_gather_scatter()(x2, edge_index)
    return _mlp(h2, W1, b1.reshape(1, D), W2, b2.reshape(1, D))


# speedup vs baseline: 1.0385x; 1.0385x over previous
import functools

import jax
import jax.numpy as jnp
from jax import lax
from jax.experimental import pallas as pl
from jax.experimental.pallas import tpu as pltpu
from jax.experimental.pallas import tpu_sc as plsc

N = 10000
E = 160000
D = 256
H = D // 2

NC = 2
NS = 16
L = 16
CHUNK = 128
NCHUNK = E // CHUNK
MAXK = -(-NCHUNK // NS)
NROW = 3
NIDX = 6
UNROLL = 6
TRIPS = -(-(MAXK + 2) // UNROLL)
ROWS_PER_TILE = 624
TAIL_ROWS = N - NS * ROWS_PER_TILE


@functools.cache
def _make_sc_gather_scatter():
    mesh = plsc.VectorSubcoreMesh(
        core_axis_name="c", subcore_axis_name="s",
        num_cores=NC, num_subcores=NS)

    @functools.partial(
        pl.kernel,
        out_type=jax.ShapeDtypeStruct((NC, N, H), jnp.float32),
        mesh=mesh,
        scratch_types=[
            pltpu.VMEM((NIDX, 2, CHUNK), jnp.int32),
            pltpu.VMEM((NROW, CHUNK, H), jnp.float32),
            pltpu.VMEM_SHARED((N, H), jnp.float32),
        ] + [pltpu.SemaphoreType.DMA] * (NIDX + 2 * NROW),
    )
    def sc_gather_scatter(xv, ei_hbm, out_hbm, idx_v, rows_v, h_sp, *sems):
        cid = lax.axis_index("c")
        tid = lax.axis_index("s")
        isem = sems[:NIDX]
        gsem = sems[NIDX:NIDX + NROW]
        ssem = sems[NIDX + NROW:]
        def unit(i):
            return i * 2 + cid

        def valid(j):
            return tid + NS * j < NCHUNK

        def off(j):
            return (tid + NS * j) * CHUNK

        def fire_idx(j, u):
            pltpu.async_copy(ei_hbm.at[0, pl.ds(off(j), CHUNK)],
                             idx_v.at[u % NIDX, 0], isem[u % NIDX])
            pltpu.async_copy(ei_hbm.at[1, pl.ds(off(j), CHUNK)],
                             idx_v.at[u % NIDX, 1], isem[u % NIDX])

        def drain_idx(j, u):
            pltpu.make_async_copy(ei_hbm.at[0, pl.ds(off(j), CHUNK)],
                                  idx_v.at[u % NIDX, 0], isem[u % NIDX]).wait()
            pltpu.make_async_copy(ei_hbm.at[1, pl.ds(off(j), CHUNK)],
                                  idx_v.at[u % NIDX, 1], isem[u % NIDX]).wait()
            for c in range(CHUNK // L):
                sl = pl.ds(c * L, L)
                idx_v[u % NIDX, 0, sl] = unit(idx_v[u % NIDX, 0, sl])

        def fire_gather(j, u):
            pltpu.async_copy(xv.at[idx_v.at[u % NIDX, 0]],
                             rows_v.at[u % NROW], gsem[u % NROW])

        def drain_gather(j, u):
            pltpu.make_async_copy(xv.at[idx_v.at[u % NIDX, 0]],
                                  rows_v.at[u % NROW], gsem[u % NROW]).wait()

        def fire_scatter(j, u):
            pltpu.async_copy(rows_v.at[u % NROW],
                             h_sp.at[idx_v.at[u % NIDX, 1]], ssem[u % NROW],
                             add=True)

        def drain_scatter(j, u):
            pltpu.make_async_copy(rows_v.at[u % NROW],
                                  h_sp.at[idx_v.at[u % NIDX, 1]],
                                  ssem[u % NROW]).wait()

        fire_idx(0, 0)
        fire_idx(1, 1)
        fire_idx(2, 2)

        def zbody(i, carry):
            for c in range(H // L):
                rows_v[0, i, pl.ds(c * L, L)] = jnp.zeros((L,), jnp.float32)
            return carry

        lax.fori_loop(0, CHUNK, zbody, 0)
        r0 = tid * ROWS_PER_TILE
        for p in range(ROWS_PER_TILE // CHUNK):
            pltpu.sync_copy(rows_v.at[0], h_sp.at[pl.ds(r0 + p * CHUNK, CHUNK)])
        rem = ROWS_PER_TILE % CHUNK
        pltpu.sync_copy(rows_v.at[0, pl.ds(0, rem)],
                        h_sp.at[pl.ds(r0 + ROWS_PER_TILE - rem, rem)])

        @pl.when(tid == NS - 1)
        def _():
            t0 = NS * ROWS_PER_TILE
            pltpu.sync_copy(rows_v.at[0, pl.ds(0, TAIL_ROWS)],
                            h_sp.at[pl.ds(t0, TAIL_ROWS)])

        plsc.subcore_barrier()

        drain_idx(0, 0)
        fire_gather(0, 0)
        drain_idx(1, 1)
        fire_gather(1, 1)

        def outer(q, carry):
            for u in range(UNROLL):
                j = q * UNROLL + u
                pl.when((j >= 1) & valid(j - 1))(
                    functools.partial(drain_scatter, j - 1, u - 1))
                pl.when(valid(j + 2))(
                    functools.partial(drain_idx, j + 2, u + 2))
                pl.when(valid(j + 2))(
                    functools.partial(fire_gather, j + 2, u + 2))
                pl.when(valid(j + 3))(
                    functools.partial(fire_idx, j + 3, u + 3))
                pl.when(valid(j))(
                    functools.partial(drain_gather, j, u))
                pl.when(valid(j))(
                    functools.partial(fire_scatter, j, u))
            return carry

        lax.fori_loop(0, TRIPS, outer, 0)
        plsc.subcore_barrier()
        pltpu.sync_copy(h_sp.at[pl.ds(r0, ROWS_PER_TILE)],
                        out_hbm.at[cid, pl.ds(r0, ROWS_PER_TILE)])

        @pl.when(tid == NS - 1)
        def _():
            t0 = NS * ROWS_PER_TILE
            pltpu.sync_copy(h_sp.at[pl.ds(t0, TAIL_ROWS)],
                            out_hbm.at[cid, pl.ds(t0, TAIL_ROWS)])

    return sc_gather_scatter


ROWS_BLK = 1000


def _mlp_body(x_ref, h2_ref, w1_ref, b1_ref, w2_ref, b2_ref, out_ref):
    h = x_ref[...] + jnp.concatenate([h2_ref[0], h2_ref[1]], axis=-1)
    z = lax.dot_general(h, w1_ref[...], (((1,), (1,)), ((), ())),
                        preferred_element_type=jnp.float32) + b1_ref[...]
    z = jnp.maximum(z, 0.0)
    out_ref[...] = lax.dot_general(z, w2_ref[...], (((1,), (1,)), ((), ())),
                                   preferred_element_type=jnp.float32) + b2_ref[...]


_mlp = pl.pallas_call(
    _mlp_body,
    grid=(N // ROWS_BLK,),
    in_specs=[
        pl.BlockSpec((ROWS_BLK, D), lambda i: (i, 0)),
        pl.BlockSpec((NC, ROWS_BLK, H), lambda i: (0, i, 0)),
        pl.BlockSpec((D, D), lambda i: (0, 0)),
        pl.BlockSpec((1, D), lambda i: (0, 0)),
        pl.BlockSpec((D, D), lambda i: (0, 0)),
        pl.BlockSpec((1, D), lambda i: (0, 0)),
    ],
    out_specs=pl.BlockSpec((ROWS_BLK, D), lambda i: (i, 0)),
    out_shape=jax.ShapeDtypeStruct((N, D), jnp.float32),
)


def kernel(x, edge_index, W1, b1, W2, b2):
    x2 = x.reshape(NC * N, H)
    h2 = _make_sc_gather_scatter()(x2, edge_index)
    return _mlp(x, h2, W1, b1.reshape(1, D), W2, b2.reshape(1, D))


# speedup vs baseline: 1.0531x; 1.0141x over previous
import functools

import jax
import jax.numpy as jnp
from jax import lax
from jax.experimental import pallas as pl
from jax.experimental.pallas import tpu as pltpu
from jax.experimental.pallas import tpu_sc as plsc

N = 10000
E = 160000
D = 256
H = D // 2

NC = 2
NS = 16
L = 16
CHUNK = 128
NCHUNK = E // CHUNK
MAXK = -(-NCHUNK // NS)
NROW = 3
NIDX = 6
UNROLL = 6
TRIPS = -(-(MAXK + 2) // UNROLL)
ROWS_PER_TILE = 624
TAIL_ROWS = N - NS * ROWS_PER_TILE


@functools.cache
def _make_sc_gather_scatter():
    mesh = plsc.VectorSubcoreMesh(
        core_axis_name="c", subcore_axis_name="s",
        num_cores=NC, num_subcores=NS)

    @functools.partial(
        pl.kernel,
        out_type=jax.ShapeDtypeStruct((NC, N, H), jnp.float32),
        mesh=mesh,
        scratch_types=[
            pltpu.VMEM((NIDX, 2, CHUNK), jnp.int32),
            pltpu.VMEM((NROW, CHUNK, H), jnp.float32),
            pltpu.VMEM_SHARED((N, H), jnp.float32),
        ] + [pltpu.SemaphoreType.DMA] * (NIDX + 2 * NROW),
    )
    def sc_gather_scatter(xv, ei_hbm, out_hbm, idx_v, rows_v, h_sp, *sems):
        cid = lax.axis_index("c")
        tid = lax.axis_index("s")
        isem = sems[:NIDX]
        gsem = sems[NIDX:NIDX + NROW]
        ssem = sems[NIDX + NROW:]
        def unit(i):
            return i * 2 + cid

        def valid(j):
            return tid + NS * j < NCHUNK

        def off(j):
            return (tid + NS * j) * CHUNK

        def fire_idx(j, u):
            pltpu.async_copy(ei_hbm.at[0, pl.ds(off(j), CHUNK)],
                             idx_v.at[u % NIDX, 0], isem[u % NIDX])
            pltpu.async_copy(ei_hbm.at[1, pl.ds(off(j), CHUNK)],
                             idx_v.at[u % NIDX, 1], isem[u % NIDX])

        def drain_idx(j, u):
            pltpu.make_async_copy(ei_hbm.at[0, pl.ds(off(j), CHUNK)],
                                  idx_v.at[u % NIDX, 0], isem[u % NIDX]).wait()
            pltpu.make_async_copy(ei_hbm.at[1, pl.ds(off(j), CHUNK)],
                                  idx_v.at[u % NIDX, 1], isem[u % NIDX]).wait()
            for c in range(CHUNK // L):
                sl = pl.ds(c * L, L)
                idx_v[u % NIDX, 0, sl] = unit(idx_v[u % NIDX, 0, sl])

        GS = CHUNK // 2

        def fire_gather(j, u):
            for g in range(2):
                pltpu.async_copy(
                    xv.at[idx_v.at[u % NIDX, 0, pl.ds(g * GS, GS)]],
                    rows_v.at[u % NROW, pl.ds(g * GS, GS)], gsem[u % NROW])

        def drain_gather(j, u):
            for g in range(2):
                pltpu.make_async_copy(
                    xv.at[idx_v.at[u % NIDX, 0, pl.ds(g * GS, GS)]],
                    rows_v.at[u % NROW, pl.ds(g * GS, GS)],
                    gsem[u % NROW]).wait()

        def fire_scatter(j, u):
            pltpu.async_copy(rows_v.at[u % NROW],
                             h_sp.at[idx_v.at[u % NIDX, 1]], ssem[u % NROW],
                             add=True)

        def drain_scatter(j, u):
            pltpu.make_async_copy(rows_v.at[u % NROW],
                                  h_sp.at[idx_v.at[u % NIDX, 1]],
                                  ssem[u % NROW]).wait()

        fire_idx(0, 0)
        fire_idx(1, 1)
        fire_idx(2, 2)

        def zbody(i, carry):
            for c in range(H // L):
                rows_v[0, i, pl.ds(c * L, L)] = jnp.zeros((L,), jnp.float32)
            return carry

        lax.fori_loop(0, CHUNK, zbody, 0)
        r0 = tid * ROWS_PER_TILE
        for p in range(ROWS_PER_TILE // CHUNK):
            pltpu.sync_copy(rows_v.at[0], h_sp.at[pl.ds(r0 + p * CHUNK, CHUNK)])
        rem = ROWS_PER_TILE % CHUNK
        pltpu.sync_copy(rows_v.at[0, pl.ds(0, rem)],
                        h_sp.at[pl.ds(r0 + ROWS_PER_TILE - rem, rem)])

        @pl.when(tid == NS - 1)
        def _():
            t0 = NS * ROWS_PER_TILE
            pltpu.sync_copy(rows_v.at[0, pl.ds(0, TAIL_ROWS)],
                            h_sp.at[pl.ds(t0, TAIL_ROWS)])

        plsc.subcore_barrier()

        drain_idx(0, 0)
        fire_gather(0, 0)
        drain_idx(1, 1)
        fire_gather(1, 1)

        def outer(q, carry):
            for u in range(UNROLL):
                j = q * UNROLL + u
                pl.when((j >= 1) & valid(j - 1))(
                    functools.partial(drain_scatter, j - 1, u - 1))
                pl.when(valid(j + 2))(
                    functools.partial(drain_idx, j + 2, u + 2))
                pl.when(valid(j + 2))(
                    functools.partial(fire_gather, j + 2, u + 2))
                pl.when(valid(j + 3))(
                    functools.partial(fire_idx, j + 3, u + 3))
                pl.when(valid(j))(
                    functools.partial(drain_gather, j, u))
                pl.when(valid(j))(
                    functools.partial(fire_scatter, j, u))
            return carry

        lax.fori_loop(0, TRIPS, outer, 0)
        plsc.subcore_barrier()
        pltpu.sync_copy(h_sp.at[pl.ds(r0, ROWS_PER_TILE)],
                        out_hbm.at[cid, pl.ds(r0, ROWS_PER_TILE)])

        @pl.when(tid == NS - 1)
        def _():
            t0 = NS * ROWS_PER_TILE
            pltpu.sync_copy(h_sp.at[pl.ds(t0, TAIL_ROWS)],
                            out_hbm.at[cid, pl.ds(t0, TAIL_ROWS)])

    return sc_gather_scatter


ROWS_BLK = 2000


def _mlp_body(x_ref, h2_ref, w1_ref, b1_ref, w2_ref, b2_ref, out_ref):
    h = x_ref[...] + jnp.concatenate([h2_ref[0], h2_ref[1]], axis=-1)
    z = lax.dot_general(h, w1_ref[...], (((1,), (1,)), ((), ())),
                        preferred_element_type=jnp.float32) + b1_ref[...]
    z = jnp.maximum(z, 0.0)
    out_ref[...] = lax.dot_general(z, w2_ref[...], (((1,), (1,)), ((), ())),
                                   preferred_element_type=jnp.float32) + b2_ref[...]


_mlp = pl.pallas_call(
    _mlp_body,
    grid=(N // ROWS_BLK,),
    in_specs=[
        pl.BlockSpec((ROWS_BLK, D), lambda i: (i, 0)),
        pl.BlockSpec((NC, ROWS_BLK, H), lambda i: (0, i, 0)),
        pl.BlockSpec((D, D), lambda i: (0, 0)),
        pl.BlockSpec((1, D), lambda i: (0, 0)),
        pl.BlockSpec((D, D), lambda i: (0, 0)),
        pl.BlockSpec((1, D), lambda i: (0, 0)),
    ],
    out_specs=pl.BlockSpec((ROWS_BLK, D), lambda i: (i, 0)),
    out_shape=jax.ShapeDtypeStruct((N, D), jnp.float32),
)


def kernel(x, edge_index, W1, b1, W2, b2):
    x2 = x.reshape(NC * N, H)
    h2 = _make_sc_gather_scatter()(x2, edge_index)
    return _mlp(x, h2, W1, b1.reshape(1, D), W2, b2.reshape(1, D))


# speedup vs baseline: 1.0533x; 1.0001x over previous
import functools

import jax
import jax.numpy as jnp
from jax import lax
from jax.experimental import pallas as pl
from jax.experimental.pallas import tpu as pltpu
from jax.experimental.pallas import tpu_sc as plsc

N = 10000
E = 160000
D = 256
H = D // 2

NC = 2
NS = 16
L = 16
CHUNK = 128
NCHUNK = E // CHUNK
MAXK = -(-NCHUNK // NS)
NROW = 3
NIDX = 6
UNROLL = 6
TRIPS = -(-(MAXK + 2) // UNROLL)
ROWS_PER_TILE = 624
TAIL_ROWS = N - NS * ROWS_PER_TILE


@functools.cache
def _make_sc_gather_scatter():
    mesh = plsc.VectorSubcoreMesh(
        core_axis_name="c", subcore_axis_name="s",
        num_cores=NC, num_subcores=NS)

    @functools.partial(
        pl.kernel,
        out_type=jax.ShapeDtypeStruct((NC, N, H), jnp.float32),
        mesh=mesh,
        scratch_types=[
            pltpu.VMEM((NIDX, 2, CHUNK), jnp.int32),
            pltpu.VMEM((NROW, CHUNK, H), jnp.float32),
            pltpu.VMEM_SHARED((N, H), jnp.float32),
        ] + [pltpu.SemaphoreType.DMA] * (NIDX + 2 * NROW),
    )
    def sc_gather_scatter(xv, ei_hbm, out_hbm, idx_v, rows_v, h_sp, *sems):
        cid = lax.axis_index("c")
        tid = lax.axis_index("s")
        isem = sems[:NIDX]
        gsem = sems[NIDX:NIDX + NROW]
        ssem = sems[NIDX + NROW:]
        def unit(i):
            return i * 2 + cid

        def valid(j):
            return tid + NS * j < NCHUNK

        def off(j):
            return (tid + NS * j) * CHUNK

        def fire_idx(j, u):
            pltpu.async_copy(ei_hbm.at[0, pl.ds(off(j), CHUNK)],
                             idx_v.at[u % NIDX, 0], isem[u % NIDX])
            pltpu.async_copy(ei_hbm.at[1, pl.ds(off(j), CHUNK)],
                             idx_v.at[u % NIDX, 1], isem[u % NIDX])

        def drain_idx(j, u):
            pltpu.make_async_copy(ei_hbm.at[0, pl.ds(off(j), CHUNK)],
                                  idx_v.at[u % NIDX, 0], isem[u % NIDX]).wait()
            pltpu.make_async_copy(ei_hbm.at[1, pl.ds(off(j), CHUNK)],
                                  idx_v.at[u % NIDX, 1], isem[u % NIDX]).wait()
            for c in range(CHUNK // L):
                sl = pl.ds(c * L, L)
                idx_v[u % NIDX, 0, sl] = unit(idx_v[u % NIDX, 0, sl])

        GS = CHUNK // 2

        def fire_gather(j, u):
            for g in range(2):
                pltpu.async_copy(
                    xv.at[idx_v.at[u % NIDX, 0, pl.ds(g * GS, GS)]],
                    rows_v.at[u % NROW, pl.ds(g * GS, GS)], gsem[u % NROW])

        def drain_gather(j, u):
            for g in range(2):
                pltpu.make_async_copy(
                    xv.at[idx_v.at[u % NIDX, 0, pl.ds(g * GS, GS)]],
                    rows_v.at[u % NROW, pl.ds(g * GS, GS)],
                    gsem[u % NROW]).wait()

        def fire_scatter(j, u):
            pltpu.async_copy(rows_v.at[u % NROW],
                             h_sp.at[idx_v.at[u % NIDX, 1]], ssem[u % NROW],
                             add=True)

        def drain_scatter(j, u):
            pltpu.make_async_copy(rows_v.at[u % NROW],
                                  h_sp.at[idx_v.at[u % NIDX, 1]],
                                  ssem[u % NROW]).wait()

        fire_idx(0, 0)
        fire_idx(1, 1)
        fire_idx(2, 2)

        def zbody(i, carry):
            for c in range(H // L):
                rows_v[0, i, pl.ds(c * L, L)] = jnp.zeros((L,), jnp.float32)
            return carry

        lax.fori_loop(0, CHUNK, zbody, 0)
        r0 = tid * ROWS_PER_TILE
        for p in range(ROWS_PER_TILE // CHUNK):
            pltpu.sync_copy(rows_v.at[0], h_sp.at[pl.ds(r0 + p * CHUNK, CHUNK)])
        rem = ROWS_PER_TILE % CHUNK
        pltpu.sync_copy(rows_v.at[0, pl.ds(0, rem)],
                        h_sp.at[pl.ds(r0 + ROWS_PER_TILE - rem, rem)])

        @pl.when(tid == NS - 1)
        def _():
            t0 = NS * ROWS_PER_TILE
            pltpu.sync_copy(rows_v.at[0, pl.ds(0, TAIL_ROWS)],
                            h_sp.at[pl.ds(t0, TAIL_ROWS)])

        plsc.subcore_barrier()

        drain_idx(0, 0)
        fire_gather(0, 0)
        drain_idx(1, 1)
        fire_gather(1, 1)

        def outer(q, carry):
            for u in range(UNROLL):
                j = q * UNROLL + u
                pl.when((j >= 1) & valid(j - 1))(
                    functools.partial(drain_scatter, j - 1, u - 1))
                pl.when(valid(j + 2))(
                    functools.partial(drain_idx, j + 2, u + 2))
                pl.when(valid(j + 2))(
                    functools.partial(fire_gather, j + 2, u + 2))
                pl.when(valid(j + 3))(
                    functools.partial(fire_idx, j + 3, u + 3))
                pl.when(valid(j))(
                    functools.partial(drain_gather, j, u))
                pl.when(valid(j))(
                    functools.partial(fire_scatter, j, u))
            return carry

        lax.fori_loop(0, TRIPS, outer, 0)
        plsc.subcore_barrier()
        pltpu.sync_copy(h_sp.at[pl.ds(r0, ROWS_PER_TILE)],
                        out_hbm.at[cid, pl.ds(r0, ROWS_PER_TILE)])

        @pl.when(tid == NS - 1)
        def _():
            t0 = NS * ROWS_PER_TILE
            pltpu.sync_copy(h_sp.at[pl.ds(t0, TAIL_ROWS)],
                            out_hbm.at[cid, pl.ds(t0, TAIL_ROWS)])

    return sc_gather_scatter


ROWS_BLK = 2000


def _mlp_body(x_ref, h2_ref, w1_ref, b1_ref, w2_ref, b2_ref, out_ref):
    h = x_ref[...] + jnp.concatenate([h2_ref[0], h2_ref[1]], axis=-1)
    hb = h.astype(jnp.bfloat16)
    z = lax.dot_general(hb, w1_ref[...].astype(jnp.bfloat16),
                        (((1,), (1,)), ((), ())),
                        preferred_element_type=jnp.float32) + b1_ref[...]
    z = jnp.maximum(z, 0.0)
    out_ref[...] = lax.dot_general(z.astype(jnp.bfloat16),
                                   w2_ref[...].astype(jnp.bfloat16),
                                   (((1,), (1,)), ((), ())),
                                   preferred_element_type=jnp.float32) + b2_ref[...]


_mlp = pl.pallas_call(
    _mlp_body,
    grid=(N // ROWS_BLK,),
    in_specs=[
        pl.BlockSpec((ROWS_BLK, D), lambda i: (i, 0)),
        pl.BlockSpec((NC, ROWS_BLK, H), lambda i: (0, i, 0)),
        pl.BlockSpec((D, D), lambda i: (0, 0)),
        pl.BlockSpec((1, D), lambda i: (0, 0)),
        pl.BlockSpec((D, D), lambda i: (0, 0)),
        pl.BlockSpec((1, D), lambda i: (0, 0)),
    ],
    out_specs=pl.BlockSpec((ROWS_BLK, D), lambda i: (i, 0)),
    out_shape=jax.ShapeDtypeStruct((N, D), jnp.float32),
)


def kernel(x, edge_index, W1, b1, W2, b2):
    x2 = x.reshape(NC * N, H)
    h2 = _make_sc_gather_scatter()(x2, edge_index)
    return _mlp(x, h2, W1, b1.reshape(1, D), W2, b2.reshape(1, D))
